# Initial kernel scaffold; baseline (speedup 1.0000x reference)
import jax
import jax.numpy as jnp
from jax.experimental import pallas as pl


def kernel(x, pos, edge_index, c1_W1, c1_b1, c1_W2, c1_b2, c2_W1, c2_b1, c2_W2, c2_b2, d_W1, d_b1, d_W2, d_b2):
    raise NotImplementedError("write your pallas kernel here")


# speedup vs baseline: 1.1029x; 1.1029x over previous
import functools

import jax
import jax.numpy as jnp
from jax.experimental import pallas as pl

N = 10000
E = 320000


def _precompute_body(h_ref, pos_ref, w1h_ref, w1p_ref, b1_ref, a_ref, q_ref):
    q = jnp.dot(pos_ref[...], w1p_ref[...], preferred_element_type=jnp.float32)
    a = jnp.dot(h_ref[...], w1h_ref[...], preferred_element_type=jnp.float32)
    a_ref[...] = a + q + b1_ref[...]
    q_ref[...] = q


def _precompute(h, pos, W1, b1):
    F = h.shape[1]
    W1h = W1[:F]
    W1p = W1[F:]
    BN = 2000
    grid = (N // BN,)
    a, q = pl.pallas_call(
        _precompute_body,
        grid=grid,
        in_specs=[
            pl.BlockSpec((BN, F), lambda i: (i, 0)),
            pl.BlockSpec((BN, 3), lambda i: (i, 0)),
            pl.BlockSpec((F, 128), lambda i: (0, 0)),
            pl.BlockSpec((3, 128), lambda i: (0, 0)),
            pl.BlockSpec((1, 128), lambda i: (0, 0)),
        ],
        out_specs=[
            pl.BlockSpec((BN, 128), lambda i: (i, 0)),
            pl.BlockSpec((BN, 128), lambda i: (i, 0)),
        ],
        out_shape=[
            jax.ShapeDtypeStruct((N, 128), jnp.float32),
            jax.ShapeDtypeStruct((N, 128), jnp.float32),
        ],
    )(h, pos, W1h, W1p, b1.reshape(1, 128))
    return a, q


def _edge_mlp_body(za_ref, zq_ref, w2_ref, b2_ref, m_ref):
    z = jnp.maximum(za_ref[...] - zq_ref[...], 0.0)
    m_ref[...] = jnp.dot(z, w2_ref[...], preferred_element_type=jnp.float32) + b2_ref[...]


def _edge_mlp(z_a, z_q, W2, b2):
    BE = 4000
    grid = (E // BE,)
    return pl.pallas_call(
        _edge_mlp_body,
        grid=grid,
        in_specs=[
            pl.BlockSpec((BE, 128), lambda i: (i, 0)),
            pl.BlockSpec((BE, 128), lambda i: (i, 0)),
            pl.BlockSpec((128, 128), lambda i: (0, 0)),
            pl.BlockSpec((1, 128), lambda i: (0, 0)),
        ],
        out_specs=pl.BlockSpec((BE, 128), lambda i: (i, 0)),
        out_shape=jax.ShapeDtypeStruct((E, 128), jnp.float32),
    )(z_a, z_q, W2, b2.reshape(1, 128))


def _decoder_body(h_ref, w1_ref, b1_ref, w2_ref, b2_ref, out_ref):
    t = jnp.maximum(
        jnp.dot(h_ref[...], w1_ref[...], preferred_element_type=jnp.float32) + b1_ref[...],
        0.0,
    )
    out_ref[...] = jnp.dot(t, w2_ref[...], preferred_element_type=jnp.float32) + b2_ref[...]


def _decoder(h, d_W1, d_b1, d_W2, d_b2):
    BN = 2000
    grid = (N // BN,)
    return pl.pallas_call(
        _decoder_body,
        grid=grid,
        in_specs=[
            pl.BlockSpec((BN, 128), lambda i: (i, 0)),
            pl.BlockSpec((128, 256), lambda i: (0, 0)),
            pl.BlockSpec((1, 256), lambda i: (0, 0)),
            pl.BlockSpec((256, 256), lambda i: (0, 0)),
            pl.BlockSpec((1, 256), lambda i: (0, 0)),
        ],
        out_specs=pl.BlockSpec((BN, 256), lambda i: (i, 0)),
        out_shape=jax.ShapeDtypeStruct((N, 256), jnp.float32),
    )(h, d_W1, d_b1.reshape(1, 256), d_W2, d_b2.reshape(1, 256))


def _pointnet_layer(h, pos, src, dst, W1, b1, W2, b2):
    a, q = _precompute(h, pos, W1, b1)
    z_a = a[src]
    z_q = q[dst]
    m = _edge_mlp(z_a, z_q, W2, b2)
    agg = jax.ops.segment_max(m, dst, num_segments=N)
    return jnp.maximum(agg, 0.0)


def kernel(x, pos, edge_index, c1_W1, c1_b1, c1_W2, c1_b2,
           c2_W1, c2_b1, c2_W2, c2_b2, d_W1, d_b1, d_W2, d_b2):
    src = edge_index[0].astype(jnp.int32)
    dst = edge_index[1].astype(jnp.int32)
    h = _pointnet_layer(x, pos, src, dst, c1_W1, c1_b1, c1_W2, c1_b2)
    h = _pointnet_layer(h, pos, src, dst, c2_W1, c2_b1, c2_W2, c2_b2)
    return _decoder(h, d_W1, d_b1, d_W2, d_b2)


# speedup vs baseline: 1.9333x; 1.7528x over previous
import functools

import jax
import jax.numpy as jnp
from jax import lax
from jax.experimental import pallas as pl
from jax.experimental.pallas import tpu as pltpu
from jax.experimental.pallas import tpu_sc as plsc

N = 10000
E = 320000
NW = 32
GC = 128
NCHUNKS = E // GC


def _precompute_body(h_ref, pos_ref, w1h_ref, w1p_ref, b1_ref, a_ref, q_ref):
    q = jnp.dot(pos_ref[...], w1p_ref[...], preferred_element_type=jnp.float32)
    a = jnp.dot(h_ref[...], w1h_ref[...], preferred_element_type=jnp.float32)
    a_ref[...] = a + q + b1_ref[...]
    q_ref[...] = q


def _precompute(h, pos, W1, b1):
    F = h.shape[1]
    W1h = W1[:F]
    W1p = W1[F:]
    BN = 2000
    grid = (N // BN,)
    a, q = pl.pallas_call(
        _precompute_body,
        grid=grid,
        in_specs=[
            pl.BlockSpec((BN, F), lambda i: (i, 0)),
            pl.BlockSpec((BN, 3), lambda i: (i, 0)),
            pl.BlockSpec((F, 128), lambda i: (0, 0)),
            pl.BlockSpec((3, 128), lambda i: (0, 0)),
            pl.BlockSpec((1, 128), lambda i: (0, 0)),
        ],
        out_specs=[
            pl.BlockSpec((BN, 128), lambda i: (i, 0)),
            pl.BlockSpec((BN, 128), lambda i: (i, 0)),
        ],
        out_shape=[
            jax.ShapeDtypeStruct((N, 128), jnp.float32),
            jax.ShapeDtypeStruct((N, 128), jnp.float32),
        ],
    )(h, pos, W1h, W1p, b1.reshape(1, 128))
    return a, q


def _edge_mlp_body(za_ref, zq_ref, w2_ref, b2_ref, m_ref):
    z = jnp.maximum(za_ref[...] - zq_ref[...], 0.0)
    m_ref[...] = jnp.dot(z, w2_ref[...], preferred_element_type=jnp.float32) + b2_ref[...]


def _edge_mlp(z_a, z_q, W2, b2):
    BE = 4000
    grid = (E // BE,)
    return pl.pallas_call(
        _edge_mlp_body,
        grid=grid,
        in_specs=[
            pl.BlockSpec((BE, 128), lambda i: (i, 0)),
            pl.BlockSpec((BE, 128), lambda i: (i, 0)),
            pl.BlockSpec((128, 128), lambda i: (0, 0)),
            pl.BlockSpec((1, 128), lambda i: (0, 0)),
        ],
        out_specs=pl.BlockSpec((BE, 128), lambda i: (i, 0)),
        out_shape=jax.ShapeDtypeStruct((E, 128), jnp.float32),
    )(z_a, z_q, W2, b2.reshape(1, 128))


def _decoder_body(h_ref, w1_ref, b1_ref, w2_ref, b2_ref, out_ref):
    t = jnp.maximum(
        jnp.dot(h_ref[...], w1_ref[...], preferred_element_type=jnp.float32) + b1_ref[...],
        0.0,
    )
    out_ref[...] = jnp.dot(t, w2_ref[...], preferred_element_type=jnp.float32) + b2_ref[...]


def _decoder(h, d_W1, d_b1, d_W2, d_b2):
    BN = 2000
    grid = (N // BN,)
    return pl.pallas_call(
        _decoder_body,
        grid=grid,
        in_specs=[
            pl.BlockSpec((BN, 128), lambda i: (i, 0)),
            pl.BlockSpec((128, 256), lambda i: (0, 0)),
            pl.BlockSpec((1, 256), lambda i: (0, 0)),
            pl.BlockSpec((256, 256), lambda i: (0, 0)),
            pl.BlockSpec((1, 256), lambda i: (0, 0)),
        ],
        out_specs=pl.BlockSpec((BN, 256), lambda i: (i, 0)),
        out_shape=jax.ShapeDtypeStruct((N, 256), jnp.float32),
    )(h, d_W1, d_b1.reshape(1, 256), d_W2, d_b2.reshape(1, 256))


def _sc_gather_body(a_hbm, q_hbm, src_hbm, dst_hbm, za_hbm, zq_hbm,
                    idx_s, idx_d, rows_a, rows_q, sem_a, sem_q):
    wid = lax.axis_index("s") * 2 + lax.axis_index("c")
    nfull = NCHUNKS // NW
    rem = NCHUNKS - nfull * NW
    ntrips = nfull + jnp.where(wid < rem, 1, 0)

    @pl.loop(0, ntrips)
    def _(t):
        base = (wid + t * NW) * GC
        pltpu.sync_copy(src_hbm.at[pl.ds(base, GC)], idx_s)
        pltpu.sync_copy(dst_hbm.at[pl.ds(base, GC)], idx_d)
        ca = pltpu.async_copy(a_hbm.at[idx_s], rows_a, sem_a)
        cq = pltpu.async_copy(q_hbm.at[idx_d], rows_q, sem_q)
        ca.wait()
        cq.wait()
        pltpu.sync_copy(rows_a, za_hbm.at[pl.ds(base, GC)])
        pltpu.sync_copy(rows_q, zq_hbm.at[pl.ds(base, GC)])


def _sc_gather(a, q, src, dst):
    mesh = plsc.VectorSubcoreMesh(core_axis_name="c", subcore_axis_name="s")
    f = pl.kernel(
        _sc_gather_body,
        out_type=[
            jax.ShapeDtypeStruct((E, 128), jnp.float32),
            jax.ShapeDtypeStruct((E, 128), jnp.float32),
        ],
        mesh=mesh,
        scratch_types=[
            pltpu.VMEM((GC,), jnp.int32),
            pltpu.VMEM((GC,), jnp.int32),
            pltpu.VMEM((GC, 128), jnp.float32),
            pltpu.VMEM((GC, 128), jnp.float32),
            pltpu.SemaphoreType.DMA,
            pltpu.SemaphoreType.DMA,
        ],
    )
    return f(a, q, src, dst)


def _pointnet_layer(h, pos, src, dst, W1, b1, W2, b2):
    a, q = _precompute(h, pos, W1, b1)
    z_a, z_q = _sc_gather(a, q, src, dst)
    m = _edge_mlp(z_a, z_q, W2, b2)
    agg = jax.ops.segment_max(m, dst, num_segments=N)
    return jnp.maximum(agg, 0.0)


def kernel(x, pos, edge_index, c1_W1, c1_b1, c1_W2, c1_b2,
           c2_W1, c2_b1, c2_W2, c2_b2, d_W1, d_b1, d_W2, d_b2):
    src = edge_index[0].astype(jnp.int32)
    dst = edge_index[1].astype(jnp.int32)
    h = _pointnet_layer(x, pos, src, dst, c1_W1, c1_b1, c1_W2, c1_b2)
    h = _pointnet_layer(h, pos, src, dst, c2_W1, c2_b1, c2_W2, c2_b2)
    return _decoder(h, d_W1, d_b1, d_W2, d_b2)


# speedup vs baseline: 2.0500x; 1.0604x over previous
import functools

import jax
import jax.numpy as jnp
from jax import lax
from jax.experimental import pallas as pl
from jax.experimental.pallas import tpu as pltpu
from jax.experimental.pallas import tpu_sc as plsc

N = 10000
E = 320000
NW = 32
GC = 128
NCHUNKS = E // GC


def _precompute_body(h_ref, pos_ref, w1h_ref, w1p_ref, b1_ref, a_ref, q_ref):
    q = jnp.dot(pos_ref[...], w1p_ref[...], preferred_element_type=jnp.float32)
    a = jnp.dot(h_ref[...], w1h_ref[...], preferred_element_type=jnp.float32)
    a_ref[...] = a + q + b1_ref[...]
    q_ref[...] = q


def _precompute(h, pos, W1, b1):
    F = h.shape[1]
    W1h = W1[:F]
    W1p = W1[F:]
    BN = 2000
    grid = (N // BN,)
    a, q = pl.pallas_call(
        _precompute_body,
        grid=grid,
        in_specs=[
            pl.BlockSpec((BN, F), lambda i: (i, 0)),
            pl.BlockSpec((BN, 3), lambda i: (i, 0)),
            pl.BlockSpec((F, 128), lambda i: (0, 0)),
            pl.BlockSpec((3, 128), lambda i: (0, 0)),
            pl.BlockSpec((1, 128), lambda i: (0, 0)),
        ],
        out_specs=[
            pl.BlockSpec((BN, 128), lambda i: (i, 0)),
            pl.BlockSpec((BN, 128), lambda i: (i, 0)),
        ],
        out_shape=[
            jax.ShapeDtypeStruct((N, 128), jnp.float32),
            jax.ShapeDtypeStruct((N, 128), jnp.float32),
        ],
    )(h, pos, W1h, W1p, b1.reshape(1, 128))
    return a, q


def _edge_mlp_body(za_ref, zq_ref, w2_ref, b2_ref, m_ref):
    z = jnp.maximum(za_ref[...] - zq_ref[...], 0.0)
    m_ref[...] = jnp.dot(z, w2_ref[...], preferred_element_type=jnp.float32) + b2_ref[...]


def _edge_mlp(z_a, z_q, W2, b2):
    BE = 4000
    grid = (E // BE,)
    return pl.pallas_call(
        _edge_mlp_body,
        grid=grid,
        in_specs=[
            pl.BlockSpec((BE, 128), lambda i: (i, 0)),
            pl.BlockSpec((BE, 128), lambda i: (i, 0)),
            pl.BlockSpec((128, 128), lambda i: (0, 0)),
            pl.BlockSpec((1, 128), lambda i: (0, 0)),
        ],
        out_specs=pl.BlockSpec((BE, 128), lambda i: (i, 0)),
        out_shape=jax.ShapeDtypeStruct((E, 128), jnp.float32),
    )(z_a, z_q, W2, b2.reshape(1, 128))


def _decoder_body(h_ref, w1_ref, b1_ref, w2_ref, b2_ref, out_ref):
    t = jnp.maximum(
        jnp.dot(h_ref[...], w1_ref[...], preferred_element_type=jnp.float32) + b1_ref[...],
        0.0,
    )
    out_ref[...] = jnp.dot(t, w2_ref[...], preferred_element_type=jnp.float32) + b2_ref[...]


def _decoder(h, d_W1, d_b1, d_W2, d_b2):
    BN = 2000
    grid = (N // BN,)
    return pl.pallas_call(
        _decoder_body,
        grid=grid,
        in_specs=[
            pl.BlockSpec((BN, 128), lambda i: (i, 0)),
            pl.BlockSpec((128, 256), lambda i: (0, 0)),
            pl.BlockSpec((1, 256), lambda i: (0, 0)),
            pl.BlockSpec((256, 256), lambda i: (0, 0)),
            pl.BlockSpec((1, 256), lambda i: (0, 0)),
        ],
        out_specs=pl.BlockSpec((BN, 256), lambda i: (i, 0)),
        out_shape=jax.ShapeDtypeStruct((N, 256), jnp.float32),
    )(h, d_W1, d_b1.reshape(1, 256), d_W2, d_b2.reshape(1, 256))


def _sc_gather_body(a_hbm, q_hbm, src_hbm, dst_hbm, za_hbm, zq_hbm,
                    idx_s, idx_d, rows_a, rows_q, sem_a, sem_q):
    wid = lax.axis_index("s") * 2 + lax.axis_index("c")
    nfull = NCHUNKS // NW
    rem = NCHUNKS - nfull * NW
    ntrips = nfull + jnp.where(wid < rem, 1, 0)

    @pl.loop(0, ntrips)
    def _(t):
        base = (wid + t * NW) * GC
        pltpu.sync_copy(src_hbm.at[pl.ds(base, GC)], idx_s)
        pltpu.sync_copy(dst_hbm.at[pl.ds(base, GC)], idx_d)
        ca = pltpu.async_copy(a_hbm.at[idx_s], rows_a, sem_a)
        cq = pltpu.async_copy(q_hbm.at[idx_d], rows_q, sem_q)
        ca.wait()
        cq.wait()
        pltpu.sync_copy(rows_a, za_hbm.at[pl.ds(base, GC)])
        pltpu.sync_copy(rows_q, zq_hbm.at[pl.ds(base, GC)])


def _sc_gather(a, q, src, dst):
    mesh = plsc.VectorSubcoreMesh(core_axis_name="c", subcore_axis_name="s")
    f = pl.kernel(
        _sc_gather_body,
        out_type=[
            jax.ShapeDtypeStruct((E, 128), jnp.float32),
            jax.ShapeDtypeStruct((E, 128), jnp.float32),
        ],
        mesh=mesh,
        scratch_types=[
            pltpu.VMEM((GC,), jnp.int32),
            pltpu.VMEM((GC,), jnp.int32),
            pltpu.VMEM((GC, 128), jnp.float32),
            pltpu.VMEM((GC, 128), jnp.float32),
            pltpu.SemaphoreType.DMA,
            pltpu.SemaphoreType.DMA,
        ],
    )
    return f(a, q, src, dst)


NB = 320
NPAD = NW * NB
SC_CH = 8000
SBUF = 8192 + 128
TRASH = SBUF
EPW = E + 128


def _sc_bucket_body(dst_hbm, midx_hbm, mloc_hbm, mcnt_hbm,
                    dbuf, mstage, lstage, cntv, sem):
    wid = lax.axis_index("s") * 2 + lax.axis_index("c")
    lo = wid * NB
    hi = lo + NB
    iota = lax.iota(jnp.int32, 16)

    def chunk(c, carry):
        gc, lc = carry
        pltpu.sync_copy(dst_hbm.at[pl.ds(c * SC_CH, SC_CH)], dbuf)

        def scan_vreg(k, lc):
            v = dbuf[pl.ds(16 * k, 16)]
            m = (v >= lo) & (v < hi)
            cs = plsc.cumsum(jnp.where(m, 1, 0))
            pos = jnp.where(m, lc + cs - 1, TRASH)
            plsc.store_scatter(mstage, [pos], c * SC_CH + 16 * k + iota)
            plsc.store_scatter(lstage, [pos], v - lo)
            return lc + cs[15]

        lc = pl.loop(0, SC_CH // 16, init_carry=lc, unroll=4)(scan_vreg)

        nfl = lc >> 7
        @pl.loop(0, nfl)
        def _(b):
            off = pl.multiple_of(wid * EPW + gc + 128 * b, 128)
            pltpu.sync_copy(mstage.at[pl.ds(128 * b, 128)],
                            midx_hbm.at[pl.ds(off, 128)])
            pltpu.sync_copy(lstage.at[pl.ds(128 * b, 128)],
                            mloc_hbm.at[pl.ds(off, 128)])
        fl = nfl << 7
        for k in range(8):
            vi = mstage[pl.ds(fl + 16 * k, 16)]
            vl = lstage[pl.ds(fl + 16 * k, 16)]
            mstage[pl.ds(16 * k, 16)] = vi
            lstage[pl.ds(16 * k, 16)] = vl
        return gc + fl, lc - fl

    gc, lc = pl.loop(0, E // SC_CH, init_carry=(jnp.int32(0), jnp.int32(0)))(chunk)

    for k in range(8):
        posn = 16 * k + iota
        vi = mstage[pl.ds(16 * k, 16)]
        vl = lstage[pl.ds(16 * k, 16)]
        mstage[pl.ds(16 * k, 16)] = jnp.where(posn < lc, vi, 0)
        lstage[pl.ds(16 * k, 16)] = jnp.where(posn < lc, vl, NB)
    nfl2 = (lc + 127) >> 7
    @pl.loop(0, nfl2)
    def _(b):
        off = pl.multiple_of(wid * EPW + gc + 128 * b, 128)
        pltpu.sync_copy(mstage.at[pl.ds(128 * b, 128)],
                        midx_hbm.at[pl.ds(off, 128)])
        pltpu.sync_copy(lstage.at[pl.ds(128 * b, 128)],
                        mloc_hbm.at[pl.ds(off, 128)])
    nblk = (gc >> 7) + nfl2
    cntv[...] = jnp.full((16,), nblk, jnp.int32)
    pltpu.sync_copy(cntv, mcnt_hbm.at[pl.ds(pl.multiple_of(wid * 16, 16), 16)])


def _sc_bucket(dst):
    mesh = plsc.VectorSubcoreMesh(core_axis_name="c", subcore_axis_name="s")
    f = pl.kernel(
        _sc_bucket_body,
        out_type=[
            jax.ShapeDtypeStruct((NW * EPW,), jnp.int32),
            jax.ShapeDtypeStruct((NW * EPW,), jnp.int32),
            jax.ShapeDtypeStruct((NW * 16,), jnp.int32),
        ],
        mesh=mesh,
        scratch_types=[
            pltpu.VMEM((SC_CH,), jnp.int32),
            pltpu.VMEM((SBUF + 16,), jnp.int32),
            pltpu.VMEM((SBUF + 16,), jnp.int32),
            pltpu.VMEM((16,), jnp.int32),
            pltpu.SemaphoreType.DMA,
        ],
        compiler_params=pltpu.CompilerParams(needs_layout_passes=False),
    )
    return f(dst)


def _sc_segmax_body(m_hbm, midx_hbm, mloc_hbm, mcnt_hbm, h_hbm,
                    idxbuf, locbuf, rows, acc, cntv, sem):
    wid = lax.axis_index("s") * 2 + lax.axis_index("c")
    pltpu.sync_copy(mcnt_hbm.at[pl.ds(pl.multiple_of(wid * 16, 16), 16)], cntv)
    nblk = cntv[...][0]

    @pl.loop(0, (NB + 1) * 8, unroll=8)
    def _(i):
        acc[pl.ds(16 * i, 16)] = jnp.zeros((16,), jnp.float32)

    @pl.loop(0, nblk)
    def _(b):
        off = pl.multiple_of(wid * EPW + 128 * b, 128)
        pltpu.sync_copy(midx_hbm.at[pl.ds(off, 128)], idxbuf)
        pltpu.sync_copy(mloc_hbm.at[pl.ds(off, 128)], locbuf)
        pltpu.async_copy(m_hbm.at[idxbuf], rows, sem).wait()

        @pl.loop(0, 8)
        def _(g):
            lvec = locbuf[pl.ds(16 * g, 16)]
            for i in range(16):
                base = lvec[i] * 128
                e = 16 * g + i
                for r in range(8):
                    cur = acc[pl.ds(base + 16 * r, 16)]
                    acc[pl.ds(base + 16 * r, 16)] = jnp.maximum(
                        cur, rows[e, pl.ds(16 * r, 16)])

    pltpu.sync_copy(acc.at[pl.ds(0, NB * 128)],
                    h_hbm.at[pl.ds(pl.multiple_of(wid * NB * 128, 128), NB * 128)])


def _sc_segmax(m, midx, mloc, mcnt):
    mesh = plsc.VectorSubcoreMesh(core_axis_name="c", subcore_axis_name="s")
    f = pl.kernel(
        _sc_segmax_body,
        out_type=jax.ShapeDtypeStruct((NPAD * 128,), jnp.float32),
        mesh=mesh,
        scratch_types=[
            pltpu.VMEM((128,), jnp.int32),
            pltpu.VMEM((128,), jnp.int32),
            pltpu.VMEM((128, 128), jnp.float32),
            pltpu.VMEM(((NB + 1) * 128,), jnp.float32),
            pltpu.VMEM((16,), jnp.int32),
            pltpu.SemaphoreType.DMA,
        ],
        compiler_params=pltpu.CompilerParams(needs_layout_passes=False),
    )
    h = f(m, midx, mloc, mcnt)
    return h.reshape(NPAD, 128)[:N]


def _pointnet_layer(h, pos, src, dst, buckets, W1, b1, W2, b2):
    a, q = _precompute(h, pos, W1, b1)
    z_a, z_q = _sc_gather(a, q, src, dst)
    m = _edge_mlp(z_a, z_q, W2, b2)
    return _sc_segmax(m, *buckets)


def kernel(x, pos, edge_index, c1_W1, c1_b1, c1_W2, c1_b2,
           c2_W1, c2_b1, c2_W2, c2_b2, d_W1, d_b1, d_W2, d_b2):
    src = edge_index[0].astype(jnp.int32)
    dst = edge_index[1].astype(jnp.int32)
    buckets = _sc_bucket(dst)
    h = _pointnet_layer(x, pos, src, dst, buckets, c1_W1, c1_b1, c1_W2, c1_b2)
    h = _pointnet_layer(h, pos, src, dst, buckets, c2_W1, c2_b1, c2_W2, c2_b2)
    return _decoder(h, d_W1, d_b1, d_W2, d_b2)
